# baseline (speedup 1.0000x reference)
import functools

import jax
import jax.numpy as jnp
from jax import lax
from jax.experimental import pallas as pl
from jax.experimental.pallas import tpu as pltpu
from jax.experimental.pallas import tpu_sc as plsc

B = 8
NB = 96
H = 224
W = 224
S = H * W
N = B * S

R = 1
CHUNK = R * W
G = N // CHUNK
NW = 32
GPW = G // NW
CPB = S // CHUNK
NVEC = CHUNK // 16

_mesh = plsc.VectorSubcoreMesh(core_axis_name="c", subcore_axis_name="s")


@functools.partial(
    pl.kernel,
    mesh=_mesh,
    out_type=jax.ShapeDtypeStruct((B, NB, H, W), jnp.float32),
    scratch_types=[
        pltpu.VMEM((GPW * CHUNK,), jnp.int32),
        pltpu.VMEM((NB, CHUNK), jnp.float32),
        pltpu.VMEM((NB, CHUNK), jnp.float32),
        pltpu.SemaphoreType.DMA,
        pltpu.SemaphoreType.DMA,
        pltpu.SemaphoreType.DMA,
    ],
    compiler_params=pltpu.CompilerParams(
        use_tc_tiling_on_sc=False, needs_layout_passes=False
    ),
)
def _sc_onehot(x_hbm, out_hbm, lbl_v, buf0, buf1, lsem, sem0, sem1):
    wid = lax.axis_index("s") * 2 + lax.axis_index("c")
    g0 = wid * GPW

    zeros = jnp.zeros((16,), jnp.float32)
    ones = jnp.ones((16,), jnp.float32)
    col_iota = lax.broadcasted_iota(jnp.int32, (16,), 0)

    pltpu.async_copy(
        x_hbm.at[pl.ds(g0 * CHUNK, GPW * CHUNK)], lbl_v, lsem
    ).wait()

    def _zero(c, _):
        for buf in (buf0, buf1):
            for j in range(CHUNK // 16):
                buf[c, pl.ds(j * 16, 16)] = zeros
        return 0

    lax.fori_loop(0, NB, _zero, 0)

    def _scatter(buf, q, val):
        for j in range(CHUNK // 16):
            off = q * CHUNK + j * 16
            lblv = lbl_v[pl.ds(off, 16)]
            plsc.store_scatter(buf, [lblv, j * 16 + col_iota], val)

    def _dmas(buf, q, sem):
        g = g0 + q
        b = g // CPB
        h0 = g % CPB
        return [
            pltpu.make_async_copy(buf, out_hbm.at[b, :, h0, :], sem)
        ]

    def _step(pair, _):
        for p, buf, sem in ((0, buf0, sem0), (1, buf1, sem1)):
            q = pair * 2 + p

            @pl.when(pair > 0)
            def _wait():
                for d in _dmas(buf, q - 2, sem):
                    d.wait()
                _scatter(buf, q - 2, zeros)

            _scatter(buf, q, ones)
            for d in _dmas(buf, q, sem):
                d.start()
        return 0

    lax.fori_loop(0, GPW // 2, _step, 0)

    for d in _dmas(buf0, GPW - 2, sem0):
        d.wait()
    for d in _dmas(buf1, GPW - 1, sem1):
        d.wait()


def kernel(x):
    return _sc_onehot(jnp.reshape(x, (N,)))


# speedup vs baseline: 4.6803x; 4.6803x over previous
import jax
import jax.numpy as jnp
from jax.experimental import pallas as pl
from jax.experimental.pallas import tpu as pltpu

NB = 96
H = 224
W = 224
C_BLK = 32


def _onehot_body(x_ref, o_ref):
    labels = x_ref[0, 0]
    c0 = pl.program_id(1) * C_BLK
    cls = c0 + jax.lax.broadcasted_iota(jnp.int32, (C_BLK, H, W), 0)
    o_ref[0] = (labels[None, :, :] == cls).astype(jnp.float32)


def kernel(x):
    grid = (x.shape[0], NB // C_BLK)
    return pl.pallas_call(
        _onehot_body,
        grid=grid,
        in_specs=[pl.BlockSpec((1, 1, H, W), lambda b, c: (b, 0, 0, 0))],
        out_specs=pl.BlockSpec((1, C_BLK, H, W), lambda b, c: (b, c, 0, 0)),
        out_shape=jax.ShapeDtypeStruct((x.shape[0], NB, H, W), jnp.float32),
        compiler_params=pltpu.CompilerParams(
            dimension_semantics=("parallel", "parallel"),
        ),
    )(x)
